# baseline (speedup 1.0000x reference)
import functools

import jax
import jax.numpy as jnp
from jax import lax
from jax.experimental import pallas as pl
from jax.experimental.pallas import tpu as pltpu
from jax.experimental.pallas import tpu_sc as plsc

N_EMB = 100000
EMB_DIM = 128
N_HASHES = 4
B = 1024
L = 200

N_TOK = B * L
NC, NS, LANES = 2, 16, 16
NW = NC * NS
BB = B // 128
NGRP = L * BB
GRP_PER_W = NGRP // NW
T = 16
Q = 128 // T
NBUF = 8
ZROWS_PER_W = GRP_PER_W * N_HASHES


@functools.partial(
    pl.kernel,
    out_type=jax.ShapeDtypeStruct((N_TOK, EMB_DIM), jnp.float32),
    mesh=plsc.VectorSubcoreMesh(
        core_axis_name="c", subcore_axis_name="s", num_cores=NC,
        num_subcores=NS),
    scratch_types=[
        pltpu.VMEM((ZROWS_PER_W, 128), jnp.int32),
        pltpu.VMEM((NBUF, N_HASHES * T, EMB_DIM), jnp.float32),
        pltpu.VMEM((NBUF, T, EMB_DIM), jnp.float32),
        pltpu.VMEM((NBUF, T), jnp.int32),
    ] + [pltpu.SemaphoreType.DMA] * 16,
)
def _bloom_sum(z_hbm, table_hbm, out_hbm, idx_v, rows_v, out_v, offs_v,
               *sems):
    wid = lax.axis_index("s") * NC + lax.axis_index("c")
    pltpu.sync_copy(z_hbm.at[pl.ds(wid * ZROWS_PER_W, ZROWS_PER_W)], idx_v)
    g0 = wid * GRP_PER_W
    gsems = list(sems[:NBUF])
    ssems = list(sems[NBUF:])

    def issue_gather(gl, q, b):
        for h in range(N_HASHES):
            pltpu.async_copy(
                table_hbm.at[idx_v.at[gl * N_HASHES + h, pl.ds(q * T, T)]],
                rows_v.at[b].at[pl.ds(h * T, T)], gsems[b])

    def wait_gather(b):
        for h in range(N_HASHES):
            pltpu.make_async_copy(
                table_hbm.at[idx_v.at[h, pl.ds(0, T)]],
                rows_v.at[b].at[pl.ds(h * T, T)], gsems[b]).wait()

    def wait_store(b):
        pltpu.make_async_copy(
            out_v.at[b], out_hbm.at[offs_v.at[b]], ssems[b]).wait()

    for _s in range(NBUF - 1):
        issue_gather(_s // Q, _s % Q, _s % NBUF)

    def grp_body(gl, carry):
        g = g0 + gl
        l = g // BB
        bb = g - l * BB
        for q in range(Q):
            b = q % NBUF
            s = gl * Q + q

            qn = (q + NBUF - 1) % Q
            gn = gl + (q + NBUF - 1) // Q

            @pl.when(gn < GRP_PER_W)
            def _():
                issue_gather(gn, qn, qn % NBUF)

            wait_gather(b)

            @pl.when(s >= NBUF)
            def _():
                wait_store(b)

            def tok_body(t, tc):
                ngroups = EMB_DIM // LANES

                def loads(d):
                    return [rows_v[b, h * T + t, pl.ds(d * LANES, LANES)]
                            for h in range(N_HASHES)]

                pipe = [loads(0), loads(1), loads(2)]
                for d in range(ngroups):
                    if d + 3 < ngroups:
                        pipe.append(loads(d + 3))
                    v0, v1, v2, v3 = pipe.pop(0)
                    out_v[b, t, pl.ds(d * LANES, LANES)] = (
                        (v0 + v1) + (v2 + v3))
                return tc

            lax.fori_loop(0, T, tok_body, 0, unroll=2)

            base = (bb * 128 + q * T) * L + l
            for jb in range(T // LANES):
                offs_v[b, pl.ds(jb * LANES, LANES)] = (
                    base + (jb * LANES + lax.iota(jnp.int32, 16)) * L)
            pltpu.async_copy(
                out_v.at[b], out_hbm.at[offs_v.at[b]], ssems[b])
        return carry

    lax.fori_loop(0, GRP_PER_W, grp_body, 0, unroll=False)
    for b in range(NBUF):
        wait_store(b)


def kernel(x, table):
    xz = (x.reshape(BB, 128, L, N_HASHES)
          .transpose(2, 0, 3, 1)
          .reshape(L * BB * N_HASHES, 128))
    out = _bloom_sum(xz, table)
    return out.reshape(B, L, EMB_DIM)


# speedup vs baseline: 1.0088x; 1.0088x over previous
import functools

import jax
import jax.numpy as jnp
from jax import lax
from jax.experimental import pallas as pl
from jax.experimental.pallas import tpu as pltpu
from jax.experimental.pallas import tpu_sc as plsc

N_EMB = 100000
EMB_DIM = 128
N_HASHES = 4
B = 1024
L = 200

N_TOK = B * L
NC, NS, LANES = 2, 16, 16
NW = NC * NS
BB = B // 128
NGRP = L * BB
GRP_PER_W = NGRP // NW
T = 32
Q = 128 // T
NBUF = 4
ZROWS_PER_W = GRP_PER_W * N_HASHES


@functools.partial(
    pl.kernel,
    out_type=jax.ShapeDtypeStruct((N_TOK, EMB_DIM), jnp.float32),
    mesh=plsc.VectorSubcoreMesh(
        core_axis_name="c", subcore_axis_name="s", num_cores=NC,
        num_subcores=NS),
    scratch_types=[
        pltpu.VMEM((ZROWS_PER_W, 128), jnp.int32),
        pltpu.VMEM((NBUF, N_HASHES * T, EMB_DIM), jnp.float32),
        pltpu.VMEM((NBUF, T, EMB_DIM), jnp.float32),
        pltpu.VMEM((NBUF, T), jnp.int32),
        pltpu.SemaphoreType.DMA,
        pltpu.SemaphoreType.DMA,
        pltpu.SemaphoreType.DMA,
        pltpu.SemaphoreType.DMA,
        pltpu.SemaphoreType.DMA,
        pltpu.SemaphoreType.DMA,
        pltpu.SemaphoreType.DMA,
        pltpu.SemaphoreType.DMA,
    ],
)
def _bloom_sum(z_hbm, table_hbm, out_hbm, idx_v, rows_v, out_v, offs_v,
               g0s, g1s, g2s, g3s, s0s, s1s, s2s, s3s):
    wid = lax.axis_index("s") * NC + lax.axis_index("c")
    pltpu.sync_copy(z_hbm.at[pl.ds(wid * ZROWS_PER_W, ZROWS_PER_W)], idx_v)
    g0 = wid * GRP_PER_W
    gsems = [g0s, g1s, g2s, g3s]
    ssems = [s0s, s1s, s2s, s3s]

    def issue_gather(gl, q, b):
        for h in range(N_HASHES):
            pltpu.async_copy(
                table_hbm.at[idx_v.at[gl * N_HASHES + h, pl.ds(q * T, T)]],
                rows_v.at[b].at[pl.ds(h * T, T)], gsems[b])

    def wait_gather(b):
        for h in range(N_HASHES):
            pltpu.make_async_copy(
                table_hbm.at[idx_v.at[h, pl.ds(0, T)]],
                rows_v.at[b].at[pl.ds(h * T, T)], gsems[b]).wait()

    def wait_store(b):
        pltpu.make_async_copy(
            out_v.at[b], out_hbm.at[offs_v.at[b]], ssems[b]).wait()

    issue_gather(0, 0, 0)
    issue_gather(0, 1, 1)
    issue_gather(0, 2, 2)

    def grp_body(gl, carry):
        g = g0 + gl
        l = g // BB
        bb = g - l * BB
        for q in range(Q):
            b = q
            s = gl * Q + q

            qn = (q + 3) % Q
            gn = gl + (q + 3) // Q

            @pl.when(gn < GRP_PER_W)
            def _():
                issue_gather(gn, qn, qn)

            wait_gather(b)

            @pl.when(s >= NBUF)
            def _():
                wait_store(b)

            def tok_body(t, tc):
                ngroups = EMB_DIM // LANES

                def loads(d):
                    return [rows_v[b, h * T + t, pl.ds(d * LANES, LANES)]
                            for h in range(N_HASHES)]

                pipe = [loads(0), loads(1), loads(2)]
                for d in range(ngroups):
                    if d + 3 < ngroups:
                        pipe.append(loads(d + 3))
                    v0, v1, v2, v3 = pipe.pop(0)
                    out_v[b, t, pl.ds(d * LANES, LANES)] = (
                        (v0 + v1) + (v2 + v3))
                return tc

            lax.fori_loop(0, T, tok_body, 0, unroll=4)

            base = (bb * 128 + q * T) * L + l
            for jb in range(T // LANES):
                offs_v[b, pl.ds(jb * LANES, LANES)] = (
                    base + (jb * LANES + lax.iota(jnp.int32, 16)) * L)
            pltpu.async_copy(
                out_v.at[b], out_hbm.at[offs_v.at[b]], ssems[b])
        return carry

    lax.fori_loop(0, GRP_PER_W, grp_body, 0, unroll=False)
    for b in range(NBUF):
        wait_store(b)


def kernel(x, table):
    xz = (x.reshape(BB, 128, L, N_HASHES)
          .transpose(2, 0, 3, 1)
          .reshape(L * BB * N_HASHES, 128))
    out = _bloom_sum(xz, table)
    return out.reshape(B, L, EMB_DIM)


# speedup vs baseline: 1.0116x; 1.0028x over previous
import functools

import jax
import jax.numpy as jnp
from jax import lax
from jax.experimental import pallas as pl
from jax.experimental.pallas import tpu as pltpu
from jax.experimental.pallas import tpu_sc as plsc

N_EMB = 100000
EMB_DIM = 128
N_HASHES = 4
B = 1024
L = 200

N_TOK = B * L
NC, NS, LANES = 2, 16, 16
NW = NC * NS
BB = B // 128
NGRP = L * BB
GRP_PER_W = NGRP // NW
T = 32
Q = 128 // T
NBUF = 4
ZROWS_PER_W = GRP_PER_W * N_HASHES


@functools.partial(
    pl.kernel,
    out_type=jax.ShapeDtypeStruct((N_TOK, EMB_DIM), jnp.float32),
    mesh=plsc.VectorSubcoreMesh(
        core_axis_name="c", subcore_axis_name="s", num_cores=NC,
        num_subcores=NS),
    scratch_types=[
        pltpu.VMEM((ZROWS_PER_W, 128), jnp.int32),
        pltpu.VMEM((NBUF, N_HASHES * T, EMB_DIM), jnp.float32),
        pltpu.VMEM((NBUF, T, EMB_DIM), jnp.float32),
        pltpu.VMEM((NBUF, T), jnp.int32),
        pltpu.SemaphoreType.DMA,
        pltpu.SemaphoreType.DMA,
        pltpu.SemaphoreType.DMA,
        pltpu.SemaphoreType.DMA,
        pltpu.SemaphoreType.DMA,
        pltpu.SemaphoreType.DMA,
        pltpu.SemaphoreType.DMA,
        pltpu.SemaphoreType.DMA,
    ],
)
def _bloom_sum(z_hbm, table_hbm, out_hbm, idx_v, rows_v, out_v, offs_v,
               g0s, g1s, g2s, g3s, s0s, s1s, s2s, s3s):
    wid = lax.axis_index("s") * NC + lax.axis_index("c")
    pltpu.sync_copy(z_hbm.at[pl.ds(wid * ZROWS_PER_W, ZROWS_PER_W)], idx_v)
    g0 = wid * GRP_PER_W
    gsems = [g0s, g1s, g2s, g3s]
    ssems = [s0s, s1s, s2s, s3s]

    def issue_gather(gl, q, b):
        for h in range(N_HASHES):
            pltpu.async_copy(
                table_hbm.at[idx_v.at[gl * N_HASHES + h, pl.ds(q * T, T)]],
                rows_v.at[b].at[pl.ds(h * T, T)], gsems[b])

    def wait_gather(b):
        for h in range(N_HASHES):
            pltpu.make_async_copy(
                table_hbm.at[idx_v.at[h, pl.ds(0, T)]],
                rows_v.at[b].at[pl.ds(h * T, T)], gsems[b]).wait()

    def wait_store(b):
        pltpu.make_async_copy(
            out_v.at[b], out_hbm.at[offs_v.at[b]], ssems[b]).wait()

    issue_gather(0, 0, 0)
    issue_gather(0, 1, 1)
    issue_gather(0, 2, 2)

    def grp_body(gl, carry):
        g = g0 + gl
        l = g // BB
        bb = g - l * BB
        for q in range(Q):
            b = q
            s = gl * Q + q

            qn = (q + 3) % Q
            gn = gl + (q + 3) // Q

            @pl.when(gn < GRP_PER_W)
            def _():
                issue_gather(gn, qn, qn)

            wait_gather(b)

            @pl.when(s >= NBUF)
            def _():
                wait_store(b)

            def tok_body(t, tc):
                ngroups = EMB_DIM // LANES

                def loads(d):
                    return [rows_v[b, h * T + t, pl.ds(d * LANES, LANES)]
                            for h in range(N_HASHES)]

                pipe = [loads(0), loads(1), loads(2)]
                for d in range(ngroups):
                    if d + 3 < ngroups:
                        pipe.append(loads(d + 3))
                    v0, v1, v2, v3 = pipe.pop(0)
                    out_v[b, t, pl.ds(d * LANES, LANES)] = (
                        (v0 + v1) + (v2 + v3))
                return tc

            lax.fori_loop(0, T, tok_body, 0, unroll=2)

            base = (bb * 128 + q * T) * L + l
            for jb in range(T // LANES):
                offs_v[b, pl.ds(jb * LANES, LANES)] = (
                    base + (jb * LANES + lax.iota(jnp.int32, 16)) * L)
            pltpu.async_copy(
                out_v.at[b], out_hbm.at[offs_v.at[b]], ssems[b])
        return carry

    lax.fori_loop(0, GRP_PER_W, grp_body, 0, unroll=False)
    for b in range(NBUF):
        wait_store(b)


def kernel(x, table):
    xz = (x.reshape(BB, 128, L, N_HASHES)
          .transpose(2, 0, 3, 1)
          .reshape(L * BB * N_HASHES, 128))
    out = _bloom_sum(xz, table)
    return out.reshape(B, L, EMB_DIM)
